# baseline (speedup 1.0000x reference)
import jax
import jax.numpy as jnp
from jax import lax
from jax.experimental import pallas as pl
from jax.experimental.pallas import tpu as pltpu
from jax.experimental.pallas import tpu_sc as plsc

ROWS, COLS = 8192, 4096
NC, NS = 2, 16
NW = NC * NS
ROWS_W = ROWS // NW
CR = 8
CC = 2048
CSPLIT = COLS // CC
NCHUNK = (ROWS_W // CR) * CSPLIT
NQUAD = NCHUNK // 4
GROUPS = CC // 128

_MESH = plsc.VectorSubcoreMesh(core_axis_name="c", subcore_axis_name="s")


def _body(x_hbm, s_hbm, o_hbm, sb0, sb1, ob0, ob1, ob2, ob3,
          sin0, sin1, xin0, xin1, xin2, xin3, out0, out1, out2, out3):
    wid = lax.axis_index("s") * NC + lax.axis_index("c")
    r0w = wid * ROWS_W
    vec8 = lax.iota(jnp.int32, 16) * 8
    zero16 = jnp.zeros((16,), jnp.float32)
    sbs = (sb0, sb1)
    obs = (ob0, ob1, ob2, ob3)
    sins = (sin0, sin1)
    xins = (xin0, xin1, xin2, xin3)
    outs = (out0, out1, out2, out3)

    def slab(c):
        r0 = r0w + (c // CSPLIT) * CR
        c0 = (c % CSPLIT) * CC
        return (pl.ds(r0, CR), pl.ds(c0, CC))

    dummy = (pl.ds(0, CR), pl.ds(0, CC))

    def start_s(c, b2):
        pltpu.async_copy(s_hbm.at[slab(c)], sbs[b2], sins[b2])

    def wait_s(b2):
        pltpu.make_async_copy(s_hbm.at[dummy], sbs[b2], sins[b2]).wait()

    def start_x(c, b4):
        pltpu.async_copy(x_hbm.at[slab(c)], obs[b4], xins[b4])

    def wait_x(b4):
        pltpu.make_async_copy(x_hbm.at[dummy], obs[b4], xins[b4]).wait()

    def start_out(c, b4):
        pltpu.async_copy(obs[b4], o_hbm.at[slab(c)], outs[b4])

    def wait_out(b4):
        pltpu.make_async_copy(obs[b4], o_hbm.at[dummy], outs[b4]).wait()

    def compute(b2, b4):
        sbuf, obuf = sbs[b2], obs[b4]

        @pl.loop(0, CR)
        def _row(rr):
            rowv = jnp.full((16,), rr, jnp.int32)

            @pl.loop(0, GROUPS, unroll=2)
            def _grp(gi):
                g0 = gi * 128
                idx = [vec8 + (g0 + p) for p in range(8)]
                s = [plsc.load_gather(sbuf, [rowv, idx[p]]) for p in range(8)]
                cnt = [jnp.full((16,), 7 - p, jnp.int32) for p in range(8)]
                for p in range(8):
                    for q in range(p + 1, 8):
                        bq = (s[p] <= s[q]).astype(jnp.int32)
                        cnt[q] = cnt[q] + bq
                        cnt[p] = cnt[p] - bq
                for p in range(8):
                    drop = cnt[p] < 4
                    plsc.store_scatter(obuf, [rowv, idx[p]], zero16,
                                       mask=drop)

    start_s(0, 0)
    start_s(1, 1)
    start_x(0, 0)
    start_x(1, 1)

    @pl.loop(0, NQUAD)
    def _quad(qi):
        for u in range(4):
            c = qi * 4 + u
            b2 = u % 2
            b4 = u
            wait_s(b2)
            wait_x(b4)
            compute(b2, b4)
            start_out(c, b4)

            @pl.when(c + 2 <= NCHUNK - 1)
            def _():
                start_s(c + 2, b2)

                @pl.when(c >= 2)
                def _():
                    wait_out((u + 2) % 4)

                start_x(c + 2, (u + 2) % 4)

    wait_out(0)
    wait_out(1)
    wait_out(2)
    wait_out(3)


@jax.jit
def _run(x, s):
    return pl.kernel(
        _body,
        out_type=jax.ShapeDtypeStruct((ROWS, COLS), jnp.float32),
        mesh=_MESH,
        scratch_types=[
            pltpu.VMEM((CR, CC), jnp.float32),
            pltpu.VMEM((CR, CC), jnp.float32),
            pltpu.VMEM((CR, CC), jnp.float32),
            pltpu.VMEM((CR, CC), jnp.float32),
            pltpu.VMEM((CR, CC), jnp.float32),
            pltpu.VMEM((CR, CC), jnp.float32),
            pltpu.SemaphoreType.DMA,
            pltpu.SemaphoreType.DMA,
            pltpu.SemaphoreType.DMA,
            pltpu.SemaphoreType.DMA,
            pltpu.SemaphoreType.DMA,
            pltpu.SemaphoreType.DMA,
            pltpu.SemaphoreType.DMA,
            pltpu.SemaphoreType.DMA,
            pltpu.SemaphoreType.DMA,
            pltpu.SemaphoreType.DMA,
        ],
        compiler_params=pltpu.CompilerParams(
            needs_layout_passes=False, use_tc_tiling_on_sc=True),
    )(x, s)


def kernel(x, score):
    return _run(x, score)


# speedup vs baseline: 1.2763x; 1.2763x over previous
import jax
import jax.numpy as jnp
from jax import lax
from jax.experimental import pallas as pl
from jax.experimental.pallas import tpu as pltpu
from jax.experimental.pallas import tpu_sc as plsc

ROWS, COLS = 8192, 4096
NC, NS = 2, 16
NW = NC * NS
ROWS_W = ROWS // NW
CR = 8
CC = 2048
CSPLIT = COLS // CC
NCHUNK = (ROWS_W // CR) * CSPLIT
NQUAD = NCHUNK // 4
GROUPS = CC // 128

_MESH = plsc.VectorSubcoreMesh(core_axis_name="c", subcore_axis_name="s")


def _body(x_hbm, s_hbm, o_hbm, sb0, sb1, ob0, ob1, ob2, ob3,
          sin0, sin1, xin0, xin1, xin2, xin3, out0, out1, out2, out3):
    wid = lax.axis_index("s") * NC + lax.axis_index("c")
    r0w = wid * ROWS_W
    vec8 = lax.iota(jnp.int32, 16) * 8
    zero16 = jnp.zeros((16,), jnp.float32)
    sbs = (sb0, sb1)
    obs = (ob0, ob1, ob2, ob3)
    sins = (sin0, sin1)
    xins = (xin0, xin1, xin2, xin3)
    outs = (out0, out1, out2, out3)

    def slab(c):
        r0 = r0w + (c // CSPLIT) * CR
        c0 = (c % CSPLIT) * CC
        return (pl.ds(r0, CR), pl.ds(c0, CC))

    dummy = (pl.ds(0, CR), pl.ds(0, CC))

    def start_s(c, b2):
        pltpu.async_copy(s_hbm.at[slab(c)], sbs[b2], sins[b2])

    def wait_s(b2):
        pltpu.make_async_copy(s_hbm.at[dummy], sbs[b2], sins[b2]).wait()

    def start_x(c, b4):
        pltpu.async_copy(x_hbm.at[slab(c)], obs[b4], xins[b4])

    def wait_x(b4):
        pltpu.make_async_copy(x_hbm.at[dummy], obs[b4], xins[b4]).wait()

    def start_out(c, b4):
        pltpu.async_copy(obs[b4], o_hbm.at[slab(c)], outs[b4])

    def wait_out(b4):
        pltpu.make_async_copy(obs[b4], o_hbm.at[dummy], outs[b4]).wait()

    def compute(b2, b4):
        sbuf, obuf = sbs[b2], obs[b4]

        @pl.loop(0, CR)
        def _row(rr):
            rowv = jnp.full((16,), rr, jnp.int32)

            @pl.loop(0, GROUPS)
            def _grp(gi):
                g0 = gi * 128
                idx = [vec8 + (g0 + p) for p in range(8)]
                s = [plsc.load_gather(sbuf, [rowv, idx[p]]) for p in range(8)]
                S = [jnp.zeros((16,), jnp.int32) for _ in range(8)]
                for p in range(8):
                    for q in range(p + 1, 8):
                        u = jnp.where(s[p] <= s[q], 1, -1)
                        S[q] = S[q] + u
                        S[p] = S[p] - u
                for p in range(8):
                    drop = S[p] < 0
                    plsc.store_scatter(obuf, [rowv, idx[p]], zero16,
                                       mask=drop)

    start_s(0, 0)
    start_s(1, 1)
    start_x(0, 0)
    start_x(1, 1)

    @pl.loop(0, NQUAD)
    def _quad(qi):
        for u in range(4):
            c = qi * 4 + u
            b2 = u % 2
            b4 = u
            wait_s(b2)
            wait_x(b4)
            compute(b2, b4)
            start_out(c, b4)

            @pl.when(c + 2 <= NCHUNK - 1)
            def _():
                start_s(c + 2, b2)

                @pl.when(c >= 2)
                def _():
                    wait_out((u + 2) % 4)

                start_x(c + 2, (u + 2) % 4)

    wait_out(0)
    wait_out(1)
    wait_out(2)
    wait_out(3)


@jax.jit
def _run(x, s):
    return pl.kernel(
        _body,
        out_type=jax.ShapeDtypeStruct((ROWS, COLS), jnp.float32),
        mesh=_MESH,
        scratch_types=[
            pltpu.VMEM((CR, CC), jnp.float32),
            pltpu.VMEM((CR, CC), jnp.float32),
            pltpu.VMEM((CR, CC), jnp.float32),
            pltpu.VMEM((CR, CC), jnp.float32),
            pltpu.VMEM((CR, CC), jnp.float32),
            pltpu.VMEM((CR, CC), jnp.float32),
            pltpu.SemaphoreType.DMA,
            pltpu.SemaphoreType.DMA,
            pltpu.SemaphoreType.DMA,
            pltpu.SemaphoreType.DMA,
            pltpu.SemaphoreType.DMA,
            pltpu.SemaphoreType.DMA,
            pltpu.SemaphoreType.DMA,
            pltpu.SemaphoreType.DMA,
            pltpu.SemaphoreType.DMA,
            pltpu.SemaphoreType.DMA,
        ],
        compiler_params=pltpu.CompilerParams(
            needs_layout_passes=False, use_tc_tiling_on_sc=True),
    )(x, s)


def kernel(x, score):
    return _run(x, score)
